# Initial kernel scaffold; baseline (speedup 1.0000x reference)
import functools

import jax
import jax.numpy as jnp
from jax.experimental import pallas as pl
from jax.experimental.pallas import tpu as pltpu

_INF = 1e9
_NEAR_T = 30.0
_FAR_T = 50.0
_HIGH_T = (2.0, 2.5, 2.5, 3.5, 2.5, 0.5, 1.0, 1.0, 0.5, 0.5)


def _body(rois_ref, lab_ref, gt_ref,
          dist_ref, high_ref, med_ref, near_ref, mid_ref, far_ref,
          idx_ref, fp_ref,
          best_d, best_i):
    i = pl.program_id(1)
    n_r = pl.num_programs(1)
    rt = rois_ref.shape[1]

    rois = rois_ref[0]
    roi_x = rois[:, 0:1]
    roi_y = rois[:, 1:2]
    valid_roi = jnp.sum(rois, axis=1, keepdims=True) != 0.0

    lab = jnp.clip(lab_ref[0], 0.0, 9.0)
    roi_grp = ((lab >= 1.0).astype(jnp.float32) + (lab >= 3.0).astype(jnp.float32)
               + (lab >= 5.0).astype(jnp.float32) + (lab >= 6.0).astype(jnp.float32)
               + (lab >= 8.0).astype(jnp.float32))

    gt = gt_ref[0]
    gt_x = gt[0:1, :]
    gt_y = gt[1:2, :]
    valid_gt = jnp.sum(gt, axis=0, keepdims=True) != 0.0
    cls = jnp.clip(jnp.floor(gt[9:10, :]), 0.0, 9.0)
    gt_grp = ((cls >= 1.0).astype(jnp.float32) + (cls >= 3.0).astype(jnp.float32)
              + (cls >= 5.0).astype(jnp.float32) + (cls >= 6.0).astype(jnp.float32)
              + (cls >= 8.0).astype(jnp.float32))

    dx = roi_x - gt_x
    dy = roi_y - gt_y
    d = jnp.sqrt(dx * dx + dy * dy)
    pv = valid_roi & valid_gt
    dist_ref[0] = jnp.where(pv, d, 0.0)

    md = jnp.where(pv & (roi_grp == gt_grp), d, _INF)
    tmin = jnp.min(md, axis=0, keepdims=True)
    ridx = (jax.lax.broadcasted_iota(jnp.int32, md.shape, 0) + i * rt
            ).astype(jnp.float32)
    tidx = jnp.min(jnp.where(md == tmin, ridx, _INF), axis=0, keepdims=True)

    @pl.when(i == 0)
    def _init():
        best_d[...] = jnp.full(best_d.shape, _INF, jnp.float32)
        best_i[...] = jnp.zeros(best_i.shape, jnp.float32)

    better = tmin < best_d[...]
    best_i[...] = jnp.where(better, tidx, best_i[...])
    best_d[...] = jnp.where(better, tmin, best_d[...])

    any_d = jnp.where(pv, d, _INF)
    rmin = jnp.min(any_d, axis=1, keepdims=True)
    fp_ref[0] = ((rmin > 4.0) & valid_roi).astype(jnp.float32)

    @pl.when(i == n_r - 1)
    def _final():
        bd = best_d[...]
        high_t = jnp.full(cls.shape, _HIGH_T[0], jnp.float32)
        for k in range(1, 10):
            high_t = jnp.where(cls == float(k), _HIGH_T[k], high_t)
        med_t = (4.0 - 1.5 * (cls >= 5.0).astype(jnp.float32)
                 - 0.5 * (cls >= 8.0).astype(jnp.float32))
        high = (bd < high_t) & valid_gt
        med = (~high) & (bd < med_t) & valid_gt
        unmatched = valid_gt & (~high) & (~med)
        grange = jnp.sqrt(gt_x * gt_x + gt_y * gt_y)
        near = unmatched & (grange < _NEAR_T)
        far = unmatched & (grange > _FAR_T)
        mid = unmatched & (~near) & (~far)
        high_ref[...] = high.astype(jnp.float32)
        med_ref[...] = med.astype(jnp.float32)
        near_ref[...] = near.astype(jnp.float32)
        mid_ref[...] = mid.astype(jnp.float32)
        far_ref[...] = far.astype(jnp.float32)
        idx_ref[...] = best_i[...]


@functools.partial(jax.jit, static_argnums=(3,), inline=True)
def _run(rois, roi_labels, gt_boxes_and_cls, interpret):
    b, r, _ = rois.shape
    g = gt_boxes_and_cls.shape[1]
    rt = 512
    r_pad = -(-r // rt) * rt
    g_pad = -(-g // 128) * 128
    n_r = r_pad // rt

    rois_p = jnp.pad(rois, ((0, 0), (0, r_pad - r), (0, 0)))
    lab_p = jnp.pad(roi_labels.astype(jnp.float32)[..., None],
                    ((0, 0), (0, r_pad - r), (0, 0)))
    gt_t = jnp.pad(jnp.swapaxes(gt_boxes_and_cls, 1, 2),
                   ((0, 0), (0, 0), (0, g_pad - g)))

    f32 = jnp.float32
    outs = pl.pallas_call(
        _body,
        grid=(b, n_r),
        in_specs=[
            pl.BlockSpec((1, rt, 9), lambda bi, ri: (bi, ri, 0)),
            pl.BlockSpec((1, rt, 1), lambda bi, ri: (bi, ri, 0)),
            pl.BlockSpec((1, 10, g_pad), lambda bi, ri: (bi, 0, 0)),
        ],
        out_specs=[
            pl.BlockSpec((1, rt, g_pad), lambda bi, ri: (bi, ri, 0)),
            pl.BlockSpec((1, g_pad), lambda bi, ri: (bi, 0)),
            pl.BlockSpec((1, g_pad), lambda bi, ri: (bi, 0)),
            pl.BlockSpec((1, g_pad), lambda bi, ri: (bi, 0)),
            pl.BlockSpec((1, g_pad), lambda bi, ri: (bi, 0)),
            pl.BlockSpec((1, g_pad), lambda bi, ri: (bi, 0)),
            pl.BlockSpec((1, g_pad), lambda bi, ri: (bi, 0)),
            pl.BlockSpec((1, rt, 1), lambda bi, ri: (bi, ri, 0)),
        ],
        out_shape=[
            jax.ShapeDtypeStruct((b, r_pad, g_pad), f32),
            jax.ShapeDtypeStruct((b, g_pad), f32),
            jax.ShapeDtypeStruct((b, g_pad), f32),
            jax.ShapeDtypeStruct((b, g_pad), f32),
            jax.ShapeDtypeStruct((b, g_pad), f32),
            jax.ShapeDtypeStruct((b, g_pad), f32),
            jax.ShapeDtypeStruct((b, g_pad), f32),
            jax.ShapeDtypeStruct((b, r_pad, 1), f32),
        ],
        scratch_shapes=[
            pltpu.VMEM((1, g_pad), f32),
            pltpu.VMEM((1, g_pad), f32),
        ],
        compiler_params=pltpu.CompilerParams(
            dimension_semantics=("arbitrary", "arbitrary")),
        interpret=interpret,
    )(rois_p, lab_p, gt_t)

    dist, high, med, near, mid, far, idxf, fp = outs
    to_bool = lambda x: x[:, :g] != 0.0
    return (dist[:, :r, :g],
            to_bool(high), to_bool(med),
            idxf[:, :g].astype(jnp.int32),
            to_bool(near), to_bool(mid), to_bool(far),
            fp[:, :r, 0] != 0.0)


def kernel(rois, roi_scores, roi_labels, gt_boxes_and_cls, batch_size):
    del roi_scores, batch_size
    return _run(rois, roi_labels, gt_boxes_and_cls, False)


# speedup vs baseline: 2.0060x; 2.0060x over previous
import functools

import jax
import jax.numpy as jnp
from jax.experimental import pallas as pl
from jax.experimental.pallas import tpu as pltpu

_INF = 1e9
_NEAR_T = 30.0
_FAR_T = 50.0
_HIGH_T = (2.0, 2.5, 2.5, 3.5, 2.5, 0.5, 1.0, 1.0, 0.5, 0.5)


def _body(rois_ref, lab_ref, gt_ref,
          dist_ref, high_ref, med_ref, near_ref, mid_ref, far_ref,
          idx_ref, fp_ref,
          best_d, best_i):
    i = pl.program_id(1)
    n_r = pl.num_programs(1)
    rt = rois_ref.shape[1]

    rois = rois_ref[0]
    roi_x = rois[:, 0:1]
    roi_y = rois[:, 1:2]
    valid_roi = jnp.sum(rois, axis=1, keepdims=True) != 0.0

    lab = jnp.clip(lab_ref[0], 0.0, 9.0)
    roi_grp = ((lab >= 1.0).astype(jnp.float32) + (lab >= 3.0).astype(jnp.float32)
               + (lab >= 5.0).astype(jnp.float32) + (lab >= 6.0).astype(jnp.float32)
               + (lab >= 8.0).astype(jnp.float32))

    gt = gt_ref[0]
    gt_x = gt[0:1, :]
    gt_y = gt[1:2, :]
    valid_gt = jnp.sum(gt, axis=0, keepdims=True) != 0.0
    cls = jnp.clip(jnp.floor(gt[9:10, :]), 0.0, 9.0)
    gt_grp = ((cls >= 1.0).astype(jnp.float32) + (cls >= 3.0).astype(jnp.float32)
              + (cls >= 5.0).astype(jnp.float32) + (cls >= 6.0).astype(jnp.float32)
              + (cls >= 8.0).astype(jnp.float32))

    dx = roi_x - gt_x
    dy = roi_y - gt_y
    d = jnp.sqrt(dx * dx + dy * dy)
    pv = valid_roi & valid_gt
    dist_ref[0] = jnp.where(pv, d, 0.0)

    md = jnp.where(pv & (roi_grp == gt_grp), d, _INF)
    tmin = jnp.min(md, axis=0, keepdims=True)
    ridx = (jax.lax.broadcasted_iota(jnp.int32, md.shape, 0) + i * rt
            ).astype(jnp.float32)
    tidx = jnp.min(jnp.where(md == tmin, ridx, _INF), axis=0, keepdims=True)

    @pl.when(i == 0)
    def _init():
        best_d[...] = jnp.full(best_d.shape, _INF, jnp.float32)
        best_i[...] = jnp.zeros(best_i.shape, jnp.float32)

    better = tmin < best_d[...]
    best_i[...] = jnp.where(better, tidx, best_i[...])
    best_d[...] = jnp.where(better, tmin, best_d[...])

    any_d = jnp.where(pv, d, _INF)
    rmin = jnp.min(any_d, axis=1, keepdims=True)
    fp_ref[0] = ((rmin > 4.0) & valid_roi).astype(jnp.float32)

    @pl.when(i == n_r - 1)
    def _final():
        bd = best_d[...]
        high_t = jnp.full(cls.shape, _HIGH_T[0], jnp.float32)
        for k in range(1, 10):
            high_t = jnp.where(cls == float(k), _HIGH_T[k], high_t)
        med_t = (4.0 - 1.5 * (cls >= 5.0).astype(jnp.float32)
                 - 0.5 * (cls >= 8.0).astype(jnp.float32))
        high = (bd < high_t) & valid_gt
        med = (~high) & (bd < med_t) & valid_gt
        unmatched = valid_gt & (~high) & (~med)
        grange = jnp.sqrt(gt_x * gt_x + gt_y * gt_y)
        near = unmatched & (grange < _NEAR_T)
        far = unmatched & (grange > _FAR_T)
        mid = unmatched & (~near) & (~far)
        high_ref[0] = high.astype(jnp.float32)
        med_ref[0] = med.astype(jnp.float32)
        near_ref[0] = near.astype(jnp.float32)
        mid_ref[0] = mid.astype(jnp.float32)
        far_ref[0] = far.astype(jnp.float32)
        idx_ref[0] = best_i[...]


@functools.partial(jax.jit, static_argnums=(3,), inline=True)
def _run(rois, roi_labels, gt_boxes_and_cls, interpret):
    b, r, _ = rois.shape
    g = gt_boxes_and_cls.shape[1]
    rt = 512
    r_pad = -(-r // rt) * rt
    g_pad = -(-g // 128) * 128
    n_r = r_pad // rt

    rois_p = jnp.pad(rois, ((0, 0), (0, r_pad - r), (0, 0)))
    lab_p = jnp.pad(roi_labels.astype(jnp.float32)[..., None],
                    ((0, 0), (0, r_pad - r), (0, 0)))
    gt_t = jnp.pad(jnp.swapaxes(gt_boxes_and_cls, 1, 2),
                   ((0, 0), (0, 0), (0, g_pad - g)))

    f32 = jnp.float32
    outs = pl.pallas_call(
        _body,
        grid=(b, n_r),
        in_specs=[
            pl.BlockSpec((1, rt, 9), lambda bi, ri: (bi, ri, 0)),
            pl.BlockSpec((1, rt, 1), lambda bi, ri: (bi, ri, 0)),
            pl.BlockSpec((1, 10, g_pad), lambda bi, ri: (bi, 0, 0)),
        ],
        out_specs=[
            pl.BlockSpec((1, rt, g_pad), lambda bi, ri: (bi, ri, 0)),
            pl.BlockSpec((1, 1, g_pad), lambda bi, ri: (bi, 0, 0)),
            pl.BlockSpec((1, 1, g_pad), lambda bi, ri: (bi, 0, 0)),
            pl.BlockSpec((1, 1, g_pad), lambda bi, ri: (bi, 0, 0)),
            pl.BlockSpec((1, 1, g_pad), lambda bi, ri: (bi, 0, 0)),
            pl.BlockSpec((1, 1, g_pad), lambda bi, ri: (bi, 0, 0)),
            pl.BlockSpec((1, 1, g_pad), lambda bi, ri: (bi, 0, 0)),
            pl.BlockSpec((1, rt, 1), lambda bi, ri: (bi, ri, 0)),
        ],
        out_shape=[
            jax.ShapeDtypeStruct((b, r_pad, g_pad), f32),
            jax.ShapeDtypeStruct((b, 1, g_pad), f32),
            jax.ShapeDtypeStruct((b, 1, g_pad), f32),
            jax.ShapeDtypeStruct((b, 1, g_pad), f32),
            jax.ShapeDtypeStruct((b, 1, g_pad), f32),
            jax.ShapeDtypeStruct((b, 1, g_pad), f32),
            jax.ShapeDtypeStruct((b, 1, g_pad), f32),
            jax.ShapeDtypeStruct((b, r_pad, 1), f32),
        ],
        scratch_shapes=[
            pltpu.VMEM((1, g_pad), f32),
            pltpu.VMEM((1, g_pad), f32),
        ],
        compiler_params=pltpu.CompilerParams(
            dimension_semantics=("arbitrary", "arbitrary")),
        interpret=interpret,
    )(rois_p, lab_p, gt_t)

    dist, high, med, near, mid, far, idxf, fp = outs
    to_bool = lambda x: x[:, 0, :g] != 0.0
    return (dist[:, :r, :g],
            to_bool(high), to_bool(med),
            idxf[:, 0, :g].astype(jnp.int32),
            to_bool(near), to_bool(mid), to_bool(far),
            fp[:, :r, 0] != 0.0)


def kernel(rois, roi_scores, roi_labels, gt_boxes_and_cls, batch_size):
    del roi_scores, batch_size
    return _run(rois, roi_labels, gt_boxes_and_cls, False)


# speedup vs baseline: 2.9460x; 1.4686x over previous
import functools

import jax
import jax.numpy as jnp
from jax.experimental import pallas as pl
from jax.experimental.pallas import tpu as pltpu

_INF = 1e9
_NEAR_T = 30.0
_FAR_T = 50.0
_HIGH_T = (2.0, 2.5, 2.5, 3.5, 2.5, 0.5, 1.0, 1.0, 0.5, 0.5)


def _body(rois_ref, lab_ref, gt_ref,
          dist_ref, high_ref, med_ref, near_ref, mid_ref, far_ref,
          idx_ref, fp_ref,
          best_d, best_i):
    i = pl.program_id(1)
    n_r = pl.num_programs(1)
    rt = rois_ref.shape[1]

    rois = rois_ref[0]
    roi_x = rois[:, 0:1]
    roi_y = rois[:, 1:2]
    valid_roi = jnp.sum(rois, axis=1, keepdims=True) != 0.0

    lab = jnp.clip(lab_ref[0], 0.0, 9.0)
    roi_grp = ((lab >= 1.0).astype(jnp.float32) + (lab >= 3.0).astype(jnp.float32)
               + (lab >= 5.0).astype(jnp.float32) + (lab >= 6.0).astype(jnp.float32)
               + (lab >= 8.0).astype(jnp.float32))
    roi_grp = jnp.where(valid_roi, roi_grp, -2.0)

    gt = gt_ref[0]
    gt_x = gt[0:1, :]
    gt_y = gt[1:2, :]
    valid_gt = jnp.sum(gt, axis=0, keepdims=True) != 0.0
    cls = jnp.clip(jnp.floor(gt[9:10, :]), 0.0, 9.0)
    gt_grp = ((cls >= 1.0).astype(jnp.float32) + (cls >= 3.0).astype(jnp.float32)
              + (cls >= 5.0).astype(jnp.float32) + (cls >= 6.0).astype(jnp.float32)
              + (cls >= 8.0).astype(jnp.float32))
    gt_grp = jnp.where(valid_gt, gt_grp, -1.0)

    dx = roi_x - gt_x
    dy = roi_y - gt_y
    d = jnp.sqrt(dx * dx + dy * dy)
    pv = valid_roi & valid_gt
    dist_ref[0] = jnp.where(pv, d, 0.0)

    md = jnp.where(roi_grp == gt_grp, d, _INF)
    tmin = jnp.min(md, axis=0, keepdims=True)
    ridx = jax.lax.broadcasted_iota(jnp.int32, md.shape, 0).astype(jnp.float32)
    lidx = jnp.min(jnp.where(md == tmin, ridx, _INF), axis=0, keepdims=True)
    tidx = lidx + (i * rt).astype(jnp.float32)

    @pl.when(i == 0)
    def _init():
        best_d[...] = jnp.full(best_d.shape, _INF, jnp.float32)
        best_i[...] = jnp.zeros(best_i.shape, jnp.float32)

    better = tmin < best_d[...]
    best_i[...] = jnp.where(better, tidx, best_i[...])
    best_d[...] = jnp.where(better, tmin, best_d[...])

    hit = pv & (d <= 4.0)
    fp = (~jnp.any(hit, axis=1, keepdims=True)) & valid_roi
    fp_ref[0] = fp.astype(jnp.float32)

    @pl.when(i == n_r - 1)
    def _final():
        bd = best_d[...]
        high_t = jnp.full(cls.shape, _HIGH_T[0], jnp.float32)
        for k in range(1, 10):
            high_t = jnp.where(cls == float(k), _HIGH_T[k], high_t)
        med_t = (4.0 - 1.5 * (cls >= 5.0).astype(jnp.float32)
                 - 0.5 * (cls >= 8.0).astype(jnp.float32))
        high = (bd < high_t) & valid_gt
        med = (~high) & (bd < med_t) & valid_gt
        unmatched = valid_gt & (~high) & (~med)
        grange = jnp.sqrt(gt_x * gt_x + gt_y * gt_y)
        near = unmatched & (grange < _NEAR_T)
        far = unmatched & (grange > _FAR_T)
        mid = unmatched & (~near) & (~far)
        high_ref[0] = high.astype(jnp.float32)
        med_ref[0] = med.astype(jnp.float32)
        near_ref[0] = near.astype(jnp.float32)
        mid_ref[0] = mid.astype(jnp.float32)
        far_ref[0] = far.astype(jnp.float32)
        idx_ref[0] = best_i[...]


@functools.partial(jax.jit, static_argnums=(3,), inline=True)
def _run(rois, roi_labels, gt_boxes_and_cls, interpret):
    b, r, _ = rois.shape
    g = gt_boxes_and_cls.shape[1]
    rt = 1000 if r % 1000 == 0 else r
    n_r = r // rt

    lab = roi_labels.astype(jnp.float32)[..., None]
    gt_t = jnp.swapaxes(gt_boxes_and_cls, 1, 2)

    f32 = jnp.float32
    outs = pl.pallas_call(
        _body,
        grid=(b, n_r),
        in_specs=[
            pl.BlockSpec((1, rt, rois.shape[2]), lambda bi, ri: (bi, ri, 0)),
            pl.BlockSpec((1, rt, 1), lambda bi, ri: (bi, ri, 0)),
            pl.BlockSpec((1, gt_t.shape[1], g), lambda bi, ri: (bi, 0, 0)),
        ],
        out_specs=[
            pl.BlockSpec((1, rt, g), lambda bi, ri: (bi, ri, 0)),
            pl.BlockSpec((1, 1, g), lambda bi, ri: (bi, 0, 0)),
            pl.BlockSpec((1, 1, g), lambda bi, ri: (bi, 0, 0)),
            pl.BlockSpec((1, 1, g), lambda bi, ri: (bi, 0, 0)),
            pl.BlockSpec((1, 1, g), lambda bi, ri: (bi, 0, 0)),
            pl.BlockSpec((1, 1, g), lambda bi, ri: (bi, 0, 0)),
            pl.BlockSpec((1, 1, g), lambda bi, ri: (bi, 0, 0)),
            pl.BlockSpec((1, rt, 1), lambda bi, ri: (bi, ri, 0)),
        ],
        out_shape=[
            jax.ShapeDtypeStruct((b, r, g), f32),
            jax.ShapeDtypeStruct((b, 1, g), f32),
            jax.ShapeDtypeStruct((b, 1, g), f32),
            jax.ShapeDtypeStruct((b, 1, g), f32),
            jax.ShapeDtypeStruct((b, 1, g), f32),
            jax.ShapeDtypeStruct((b, 1, g), f32),
            jax.ShapeDtypeStruct((b, 1, g), f32),
            jax.ShapeDtypeStruct((b, r, 1), f32),
        ],
        scratch_shapes=[
            pltpu.VMEM((1, g), f32),
            pltpu.VMEM((1, g), f32),
        ],
        compiler_params=pltpu.CompilerParams(
            dimension_semantics=("arbitrary", "arbitrary")),
        interpret=interpret,
    )(rois, lab, gt_t)

    dist, high, med, near, mid, far, idxf, fp = outs
    to_bool = lambda x: x[:, 0, :] != 0.0
    return (dist,
            to_bool(high), to_bool(med),
            idxf[:, 0, :].astype(jnp.int32),
            to_bool(near), to_bool(mid), to_bool(far),
            fp[:, :, 0] != 0.0)


def kernel(rois, roi_scores, roi_labels, gt_boxes_and_cls, batch_size):
    del roi_scores, batch_size
    return _run(rois, roi_labels, gt_boxes_and_cls, False)
